# Initial kernel scaffold; baseline (speedup 1.0000x reference)
import jax
import jax.numpy as jnp
from jax.experimental import pallas as pl


def kernel(idx, targets, W):
    raise NotImplementedError("write your pallas kernel here")


# speedup vs baseline: 1.4095x; 1.4095x over previous
import functools

import jax
import jax.numpy as jnp
from jax import lax
from jax.experimental import pallas as pl
from jax.experimental.pallas import tpu as pltpu
from jax.experimental.pallas import tpu_sc as plsc

NC = 2
NS = 16
NW = NC * NS
LANES = 16


def _lse_body(w_ref, out_ref):
    w = w_ref[...]
    m = jnp.max(w, axis=1)
    e = jnp.exp(w - m[:, None])
    out_ref[...] = m + jnp.log(jnp.sum(e, axis=1))


def _make_sc_kernel(n_tokens, vocab, chunk):
    tok_per_tile = n_tokens // NW
    n_chunks = tok_per_tile // chunk
    mesh = plsc.VectorSubcoreMesh(core_axis_name="c", subcore_axis_name="s")

    @functools.partial(
        pl.kernel,
        out_type=(
            jax.ShapeDtypeStruct((n_tokens, vocab), jnp.float32),
            jax.ShapeDtypeStruct((NW, LANES), jnp.float32),
        ),
        mesh=mesh,
        scratch_types=[
            pltpu.VMEM((tok_per_tile,), jnp.int32),
            pltpu.VMEM((tok_per_tile,), jnp.int32),
            pltpu.VMEM((vocab,), jnp.float32),
            pltpu.VMEM((chunk, vocab), jnp.float32),
            pltpu.VMEM((LANES,), jnp.float32),
            pltpu.SemaphoreType.DMA,
        ],
        compiler_params=pltpu.CompilerParams(
            use_tc_tiling_on_sc=False, needs_layout_passes=False
        ),
    )
    def sc_kernel(w_hbm, idx_hbm, tgt_hbm, lse_hbm, logits_hbm, partial_hbm,
                  idx_v, tgt_v, lse_v, rows_v, acc_v, sem):
        c = lax.axis_index("c")
        s = lax.axis_index("s")
        wid = s * NC + c
        base = wid * tok_per_tile
        pltpu.sync_copy(idx_hbm.at[pl.ds(base, tok_per_tile)], idx_v)
        pltpu.sync_copy(tgt_hbm.at[pl.ds(base, tok_per_tile)], tgt_v)
        pltpu.sync_copy(lse_hbm, lse_v)

        def chunk_body(ci, acc):
            off = ci * chunk
            pltpu.async_copy(
                w_hbm.at[idx_v.at[pl.ds(off, chunk)]], rows_v, sem
            ).wait()

            def grp(j, acc):
                lid = lax.iota(jnp.int32, LANES) + j * LANES
                tgt16 = tgt_v[pl.ds(off + j * LANES, LANES)]
                idx16 = idx_v[pl.ds(off + j * LANES, LANES)]
                tgt_vals = plsc.load_gather(rows_v, [lid, tgt16])
                lse_vals = plsc.load_gather(lse_v, [idx16])
                return acc + (lse_vals - tgt_vals)

            acc = lax.fori_loop(0, chunk // LANES, grp, acc)
            pltpu.sync_copy(rows_v, logits_hbm.at[pl.ds(base + off, chunk)])
            return acc

        acc = lax.fori_loop(0, n_chunks, chunk_body,
                            jnp.zeros((LANES,), jnp.float32))
        acc_v[...] = acc
        pltpu.sync_copy(acc_v, partial_hbm.at[wid])

    return sc_kernel


@jax.jit
def kernel(idx, targets, W):
    b, t = idx.shape
    vocab = W.shape[0]
    n = b * t

    lse = pl.pallas_call(
        _lse_body,
        out_shape=jax.ShapeDtypeStruct((vocab,), jnp.float32),
    )(W)

    flat_idx = idx.reshape(n).astype(jnp.int32)
    flat_tgt = targets.reshape(n).astype(jnp.int32)

    sc = _make_sc_kernel(n, vocab, chunk=64)
    logits_flat, partials = sc(W, flat_idx, flat_tgt, lse)

    loss = jnp.sum(partials) / n
    return logits_flat.reshape(b, t, vocab), loss
